# baseline (speedup 1.0000x reference)
import functools

import jax
import jax.numpy as jnp
from jax import lax
from jax.experimental import pallas as pl
from jax.experimental.pallas import tpu as pltpu
from jax.experimental.pallas import tpu_sc as plsc

TM = 1024
NK = 1024


_CHUNK_SPLIT_A, _SPLIT_COL_A = 2, 688
_CHUNK_SPLIT_B, _SPLIT_COL_B = 5, 352


def _rne_bf16(v):
    b = lax.bitcast_convert_type(v, jnp.uint32)
    b = b + jnp.uint32(0x7FFF) + ((b >> 16) & jnp.uint32(1))
    b = b & jnp.uint32(0xFFFF0000)
    return lax.bitcast_convert_type(b, jnp.float32)


def _argmin_body(z_ref, e_ref, zsq_ref, esq_ref, o_ref, rmin_ref):
    j = pl.program_id(1)
    zb = z_ref[...]
    eb = e_ref[...]

    @pl.when(j == 0)
    def _():
        rmin_ref[...] = jnp.full((zb.shape[0], 1), jnp.inf, jnp.float32)
        o_ref[...] = jnp.zeros((zb.shape[0], 1), jnp.int32)

    zsq = zsq_ref[...]
    esq = esq_ref[...].reshape(1, eb.shape[0])
    m = lax.dot_general(
        zb, eb, (((1,), (1,)), ((), ())),
        preferred_element_type=jnp.float32)
    d = (zsq + esq) - 2.0 * m
    nk = d.shape[1]
    iota = lax.broadcasted_iota(jnp.int32, d.shape, 1)

    def combine(lmin, larg):
        rmin = rmin_ref[...]
        upd = lmin < rmin
        rmin_ref[...] = jnp.where(upd, lmin, rmin)
        o_ref[...] = jnp.where(upd, larg, o_ref[...])

    def masked_minarg(mask):
        lmin = jnp.min(jnp.where(mask, d, jnp.inf), axis=1, keepdims=True)
        larg = jnp.min(jnp.where(mask & (d == lmin), iota, nk),
                       axis=1, keepdims=True) + j * nk
        return lmin, larg

    is_split = (j == _CHUNK_SPLIT_A) | (j == _CHUNK_SPLIT_B)
    split_col = jnp.where(j == _CHUNK_SPLIT_A, _SPLIT_COL_A, _SPLIT_COL_B)

    @pl.when(jnp.logical_not(is_split))
    def _():
        lmin = jnp.min(d, axis=1, keepdims=True)
        larg = jnp.min(jnp.where(d == lmin, iota, nk),
                       axis=1, keepdims=True) + j * nk
        combine(lmin, larg)

    @pl.when(is_split)
    def _():
        mask_a = iota < split_col
        lmin, larg = masked_minarg(mask_a)
        combine(lmin, larg)
        rmin_ref[...] = _rne_bf16(rmin_ref[...])
        lmin, larg = masked_minarg(jnp.logical_not(mask_a))
        combine(lmin, larg)


def _argmin_call(z_flat, embedding, zsq, esq3):
    M, C = z_flat.shape
    N = embedding.shape[0]
    return pl.pallas_call(
        _argmin_body,
        grid=(M // TM, N // NK),
        in_specs=[
            pl.BlockSpec((TM, C), lambda i, j: (i, 0)),
            pl.BlockSpec((NK, C), lambda i, j: (j, 0)),
            pl.BlockSpec((TM, 1), lambda i, j: (i, 0)),
            pl.BlockSpec((1, 1, NK), lambda i, j: (j, 0, 0)),
        ],
        out_specs=pl.BlockSpec((TM, 1), lambda i, j: (i, 0)),
        out_shape=jax.ShapeDtypeStruct((M, 1), jnp.int32),
        scratch_shapes=[
            pltpu.VMEM((TM, 1), jnp.float32),
        ],
    )(z_flat, embedding, zsq, esq3)


def _sc_gather(embedding, idx):
    M = idx.shape[0]
    D = embedding.shape[1]
    info = plsc.get_sparse_core_info()
    nc, ns = info.num_cores, info.num_subcores
    nw = nc * ns
    bpw = M // nw
    ch = 128
    nch = bpw // ch
    mesh = plsc.VectorSubcoreMesh(core_axis_name="c", subcore_axis_name="s")

    @functools.partial(
        pl.kernel, mesh=mesh,
        out_type=jax.ShapeDtypeStruct((M, D), jnp.float32),
        scratch_types=(
            [pltpu.VMEM((ch,), jnp.int32) for _ in range(nch)]
            + [pltpu.VMEM((ch, D), jnp.float32) for _ in range(2)]
            + [pltpu.SemaphoreType.DMA for _ in range(2)]
        ),
    )
    def gk(emb_hbm, idx_hbm, out_hbm, *refs):
        idx_v = refs[:nch]
        bufs = refs[nch:nch + 2]
        sems = refs[nch + 2:nch + 4]
        wid = lax.axis_index("s") * nc + lax.axis_index("c")
        base = wid * bpw
        for c in range(nch):
            pltpu.sync_copy(idx_hbm.at[pl.ds(base + c * ch, ch)], idx_v[c])
        cps = [None, None]
        cps[0] = pltpu.async_copy(emb_hbm.at[idx_v[0]], bufs[0], sems[0])
        for c in range(nch):
            if c + 1 < nch:
                cps[(c + 1) % 2] = pltpu.async_copy(
                    emb_hbm.at[idx_v[c + 1]], bufs[(c + 1) % 2],
                    sems[(c + 1) % 2])
            cps[c % 2].wait()
            pltpu.sync_copy(bufs[c % 2], out_hbm.at[pl.ds(base + c * ch, ch)])

    return gk(embedding, idx)


def kernel(z, embedding):
    z = z.astype(jnp.float32)
    S, B, C = z.shape
    N = embedding.shape[0]
    z_flat = z.reshape(S * B, C)
    zsq = jnp.sum(z_flat ** 2, axis=1, keepdims=True)
    esq3 = jnp.sum(embedding ** 2, axis=1).reshape(N // NK, 1, NK)
    idx = _argmin_call(z_flat, embedding, zsq, esq3).reshape(S * B)
    zq = _sc_gather(embedding, idx)
    return zq.reshape(S, B, C)


# speedup vs baseline: 1.3664x; 1.3664x over previous
import functools

import jax
import jax.numpy as jnp
from jax import lax
from jax.experimental import pallas as pl
from jax.experimental.pallas import tpu as pltpu
from jax.experimental.pallas import tpu_sc as plsc

TM = 1024
NK = 2736


def _rne_bf16(v):
    b = lax.bitcast_convert_type(v, jnp.uint32)
    b = b + jnp.uint32(0x7FFF) + ((b >> 16) & jnp.uint32(1))
    b = b & jnp.uint32(0xFFFF0000)
    return lax.bitcast_convert_type(b, jnp.float32)


def _argmin_body(z_ref, e_ref, zsq_ref, esq_ref, o_ref, rmin_ref):
    j = pl.program_id(1)
    zb = z_ref[...]
    eb = e_ref[...]

    @pl.when(j == 0)
    def _():
        rmin_ref[...] = jnp.full((zb.shape[0], 1), jnp.inf, jnp.float32)
        o_ref[...] = jnp.zeros((zb.shape[0], 1), jnp.int32)

    zsq = zsq_ref[...]
    esq = esq_ref[...].reshape(1, eb.shape[0])
    m = lax.dot_general(
        zb, eb, (((1,), (1,)), ((), ())),
        preferred_element_type=jnp.float32)
    d = (zsq + esq) - 2.0 * m
    nk = d.shape[1]
    iota = lax.broadcasted_iota(jnp.int32, d.shape, 1)
    lmin = jnp.min(d, axis=1, keepdims=True)
    larg = jnp.min(jnp.where(d == lmin, iota, nk),
                   axis=1, keepdims=True) + j * nk
    rmin = rmin_ref[...]
    upd = lmin < rmin
    rmin_ref[...] = _rne_bf16(jnp.where(upd, lmin, rmin))
    o_ref[...] = jnp.where(upd, larg, o_ref[...])


def _argmin_call(z_flat, emb_pad, zsq, esq3):
    M, C = z_flat.shape
    NP = emb_pad.shape[0]
    return pl.pallas_call(
        _argmin_body,
        grid=(M // TM, NP // NK),
        in_specs=[
            pl.BlockSpec((TM, C), lambda i, j: (i, 0)),
            pl.BlockSpec((NK, C), lambda i, j: (j, 0)),
            pl.BlockSpec((TM, 1), lambda i, j: (i, 0)),
            pl.BlockSpec((1, 1, NK), lambda i, j: (j, 0, 0)),
        ],
        out_specs=pl.BlockSpec((TM, 1), lambda i, j: (i, 0)),
        out_shape=jax.ShapeDtypeStruct((M, 1), jnp.int32),
        scratch_shapes=[
            pltpu.VMEM((TM, 1), jnp.float32),
        ],
    )(z_flat, emb_pad, zsq, esq3)


def _sc_gather(embedding, idx):
    M = idx.shape[0]
    D = embedding.shape[1]
    info = plsc.get_sparse_core_info()
    nc, ns = info.num_cores, info.num_subcores
    nw = nc * ns
    bpw = M // nw
    ch = 128
    nch = bpw // ch
    mesh = plsc.VectorSubcoreMesh(core_axis_name="c", subcore_axis_name="s")

    @functools.partial(
        pl.kernel, mesh=mesh,
        out_type=jax.ShapeDtypeStruct((M, D), jnp.float32),
        scratch_types=(
            [pltpu.VMEM((ch,), jnp.int32) for _ in range(nch)]
            + [pltpu.VMEM((ch, D), jnp.float32) for _ in range(2)]
            + [pltpu.SemaphoreType.DMA for _ in range(2)]
        ),
    )
    def gk(emb_hbm, idx_hbm, out_hbm, *refs):
        idx_v = refs[:nch]
        bufs = refs[nch:nch + 2]
        sems = refs[nch + 2:nch + 4]
        wid = lax.axis_index("s") * nc + lax.axis_index("c")
        base = wid * bpw
        for c in range(nch):
            pltpu.sync_copy(idx_hbm.at[pl.ds(base + c * ch, ch)], idx_v[c])
        cps = [None, None]
        cps[0] = pltpu.async_copy(emb_hbm.at[idx_v[0]], bufs[0], sems[0])
        for c in range(nch):
            if c + 1 < nch:
                cps[(c + 1) % 2] = pltpu.async_copy(
                    emb_hbm.at[idx_v[c + 1]], bufs[(c + 1) % 2],
                    sems[(c + 1) % 2])
            cps[c % 2].wait()
            pltpu.sync_copy(bufs[c % 2], out_hbm.at[pl.ds(base + c * ch, ch)])

    return gk(embedding, idx)


def kernel(z, embedding):
    z = z.astype(jnp.float32)
    S, B, C = z.shape
    N = embedding.shape[0]
    NP = 3 * NK
    z_flat = z.reshape(S * B, C)
    zsq = jnp.sum(z_flat ** 2, axis=1, keepdims=True)
    esq = jnp.sum(embedding ** 2, axis=1)
    emb_pad = jnp.concatenate(
        [embedding, jnp.zeros((NP - N, C), jnp.float32)], axis=0)
    esq3 = jnp.concatenate(
        [esq, jnp.full((NP - N,), jnp.inf, jnp.float32)]).reshape(3, 1, NK)
    idx = _argmin_call(z_flat, emb_pad, zsq, esq3).reshape(S * B)
    zq = _sc_gather(embedding, idx)
    return zq.reshape(S, B, C)


# speedup vs baseline: 1.4344x; 1.0498x over previous
import functools

import jax
import jax.numpy as jnp
from jax import lax
from jax.experimental import pallas as pl
from jax.experimental.pallas import tpu as pltpu
from jax.experimental.pallas import tpu_sc as plsc

TM = 2048
NK = 2736


def _rne_bf16(v):
    b = lax.bitcast_convert_type(v, jnp.uint32)
    b = b + jnp.uint32(0x7FFF) + ((b >> 16) & jnp.uint32(1))
    b = b & jnp.uint32(0xFFFF0000)
    return lax.bitcast_convert_type(b, jnp.float32)


def _argmin_body(z_ref, e_ref, zsq_ref, esq_ref, o_ref, rmin_ref):
    j = pl.program_id(1)
    zb = z_ref[...]
    eb = e_ref[...]

    @pl.when(j == 0)
    def _():
        rmin_ref[...] = jnp.full((zb.shape[0], 1), jnp.inf, jnp.float32)
        o_ref[...] = jnp.zeros((zb.shape[0], 1), jnp.int32)

    zsq = zsq_ref[...]
    esq = esq_ref[...].reshape(1, eb.shape[0])
    m = lax.dot_general(
        zb, eb, (((1,), (1,)), ((), ())),
        preferred_element_type=jnp.float32)
    d = (zsq + esq) - 2.0 * m
    nk = d.shape[1]
    iota = lax.broadcasted_iota(jnp.int32, d.shape, 1)
    lmin = jnp.min(d, axis=1, keepdims=True)
    larg = jnp.min(jnp.where(d == lmin, iota, nk),
                   axis=1, keepdims=True) + j * nk
    rmin = rmin_ref[...]
    upd = lmin < rmin
    rmin_ref[...] = _rne_bf16(jnp.where(upd, lmin, rmin))
    o_ref[...] = jnp.where(upd, larg, o_ref[...])


def _argmin_call(z_flat, emb_pad, zsq, esq3):
    M, C = z_flat.shape
    NP = emb_pad.shape[0]
    return pl.pallas_call(
        _argmin_body,
        grid=(M // TM, NP // NK),
        in_specs=[
            pl.BlockSpec((TM, C), lambda i, j: (i, 0)),
            pl.BlockSpec((NK, C), lambda i, j: (j, 0)),
            pl.BlockSpec((TM, 1), lambda i, j: (i, 0)),
            pl.BlockSpec((1, 1, NK), lambda i, j: (j, 0, 0)),
        ],
        out_specs=pl.BlockSpec((TM, 1), lambda i, j: (i, 0)),
        out_shape=jax.ShapeDtypeStruct((M, 1), jnp.int32),
        scratch_shapes=[
            pltpu.VMEM((TM, 1), jnp.float32),
        ],
    )(z_flat, emb_pad, zsq, esq3)


def _sc_gather(embedding, idx):
    M = idx.shape[0]
    D = embedding.shape[1]
    info = plsc.get_sparse_core_info()
    nc, ns = info.num_cores, info.num_subcores
    nw = nc * ns
    bpw = M // nw
    ch = 128
    nch = bpw // ch
    mesh = plsc.VectorSubcoreMesh(core_axis_name="c", subcore_axis_name="s")

    @functools.partial(
        pl.kernel, mesh=mesh,
        out_type=jax.ShapeDtypeStruct((M, D), jnp.float32),
        scratch_types=(
            [pltpu.VMEM((ch,), jnp.int32) for _ in range(nch)]
            + [pltpu.VMEM((ch, D), jnp.float32) for _ in range(2)]
            + [pltpu.SemaphoreType.DMA for _ in range(2)]
        ),
    )
    def gk(emb_hbm, idx_hbm, out_hbm, *refs):
        idx_v = refs[:nch]
        bufs = refs[nch:nch + 2]
        sems = refs[nch + 2:nch + 4]
        wid = lax.axis_index("s") * nc + lax.axis_index("c")
        base = wid * bpw
        for c in range(nch):
            pltpu.sync_copy(idx_hbm.at[pl.ds(base + c * ch, ch)], idx_v[c])
        cps = [None, None]
        cps[0] = pltpu.async_copy(emb_hbm.at[idx_v[0]], bufs[0], sems[0])
        for c in range(nch):
            if c + 1 < nch:
                cps[(c + 1) % 2] = pltpu.async_copy(
                    emb_hbm.at[idx_v[c + 1]], bufs[(c + 1) % 2],
                    sems[(c + 1) % 2])
            cps[c % 2].wait()
            pltpu.sync_copy(bufs[c % 2], out_hbm.at[pl.ds(base + c * ch, ch)])

    return gk(embedding, idx)


def kernel(z, embedding):
    z = z.astype(jnp.float32)
    S, B, C = z.shape
    N = embedding.shape[0]
    NP = 3 * NK
    z_flat = z.reshape(S * B, C)
    zsq = jnp.sum(z_flat ** 2, axis=1, keepdims=True)
    esq = jnp.sum(embedding ** 2, axis=1)
    emb_pad = jnp.concatenate(
        [embedding, jnp.zeros((NP - N, C), jnp.float32)], axis=0)
    esq3 = jnp.concatenate(
        [esq, jnp.full((NP - N,), jnp.inf, jnp.float32)]).reshape(3, 1, NK)
    idx = _argmin_call(z_flat, emb_pad, zsq, esq3).reshape(S * B)
    zq = _sc_gather(embedding, idx)
    return zq.reshape(S, B, C)
